# baseline (speedup 1.0000x reference)
import functools

import jax
import jax.numpy as jnp
from jax.experimental import pallas as pl
from jax.experimental.pallas import tpu as pltpu

_F = 7
_P = (_F - 1) // 2
_EPS = 1e-5

def _cell_kernel(
    H, W, Ch, MT,
    xe_ref,
    ihw_ref,
    w3_ref,
    bg_ref,
    w7_ref,
    g1g_ref, g1b_ref,
    g2g_ref, g2b_ref,
    bwe_ref,
    g3g_ref, g3b_ref,
    out_ref,
    ew_ref,
    big_ref,
):
    HW = H * W
    PW = _P * W
    Cin = xe_ref.shape[2] - Ch
    inv_n = 1.0 / float(HW * Ch)
    xe = xe_ref[0]

    big_ref[...] = jnp.dot(xe, w3_ref[...],
                           preferred_element_type=jnp.float32)

    bg = bg_ref[...]
    v1 = big_ref[:, 0:Ch]
    v2 = big_ref[:, Ch:2 * Ch]
    s1c = jnp.sum(v1, axis=0, keepdims=True)
    q1c = jnp.sum(v1 * v1, axis=0, keepdims=True)
    s2c = jnp.sum(v2, axis=0, keepdims=True)
    q2c = jnp.sum(v2 * v2, axis=0, keepdims=True)
    b1 = bg[:, 0:Ch]
    b2 = bg[:, Ch:2 * Ch]
    s1 = jnp.sum(s1c + float(HW) * b1, axis=1, keepdims=True)
    s2 = jnp.sum(s2c + float(HW) * b2, axis=1, keepdims=True)
    q1 = jnp.sum(q1c + 2.0 * b1 * s1c + float(HW) * b1 * b1,
                 axis=1, keepdims=True)
    q2 = jnp.sum(q2c + 2.0 * b2 * s2c + float(HW) * b2 * b2,
                 axis=1, keepdims=True)
    mu1 = s1 * inv_n
    var1 = q1 * inv_n - mu1 * mu1
    r1 = jax.lax.rsqrt(var1 + _EPS)
    mu2 = s2 * inv_n
    var2 = q2 * inv_n - mu2 * mu2
    r2 = jax.lax.rsqrt(var2 + _EPS)
    c1 = r1 * g1g_ref[...]
    c2 = r2 * g2g_ref[...]
    o12 = (g1b_ref[...] + g2b_ref[...]
           + (b1 - mu1) * c1 + (b2 - mu2) * c2)

    ew_ref[0:PW, :] = jnp.zeros((PW, _F * Ch), jnp.bfloat16)
    ew_ref[PW + HW:, :] = jnp.zeros((PW, _F * Ch), jnp.bfloat16)

    e2 = xe[:, Cin:Cin + Ch].astype(jnp.float32)
    pcol = jax.lax.broadcasted_iota(jnp.int32, (HW, Ch), 0) & (W - 1)
    for kw in range(_F):
        d = kw - _P
        rolled = pltpu.roll(e2, (-d) % HW, axis=0) if d else e2
        if d > 0:
            blk = jnp.where(pcol <= (W - 1 - d), rolled, 0.0)
        elif d < 0:
            blk = jnp.where(pcol >= (-d), rolled, 0.0)
        else:
            blk = rolled
        ew_ref[PW:PW + HW, kw * Ch:(kw + 1) * Ch] = blk.astype(jnp.bfloat16)

    bwe = bwe_ref[...]
    s3 = jnp.zeros((1, Ch), jnp.float32)
    q3 = jnp.zeros((1, Ch), jnp.float32)
    for m0 in range(0, HW, MT):
        acc = jnp.dot(ew_ref[m0:m0 + MT, :], w7_ref[0],
                      preferred_element_type=jnp.float32)
        for kh in range(1, _F):
            acc = acc + jnp.dot(ew_ref[m0 + kh * W:m0 + kh * W + MT, :],
                                w7_ref[kh], preferred_element_type=jnp.float32)
        cand = jnp.maximum(acc + big_ref[m0:m0 + MT, 2 * Ch:3 * Ch] + bwe, 0.0)
        g = jax.nn.sigmoid(big_ref[m0:m0 + MT, 0:Ch] * c1
                           + big_ref[m0:m0 + MT, Ch:2 * Ch] * c2 + o12)
        et = e2[m0:m0 + MT, :]
        pre = g * cand + et - g * et
        big_ref[m0:m0 + MT, 0:Ch] = pre
        s3 = s3 + jnp.sum(pre, axis=0, keepdims=True)
        q3 = q3 + jnp.sum(pre * pre, axis=0, keepdims=True)

    s3t = jnp.sum(s3, axis=1, keepdims=True)
    q3t = jnp.sum(q3, axis=1, keepdims=True)
    mu3 = s3t * inv_n
    var3 = q3t * inv_n - mu3 * mu3
    r3 = jax.lax.rsqrt(var3 + _EPS)
    c3 = r3 * g3g_ref[...]
    o3 = g3b_ref[...] - mu3 * c3
    outt = jnp.maximum(big_ref[:, 0:Ch] * c3 + o3, 0.0)
    out_ref[0] = jax.lax.dot_general(
        outt.astype(jnp.bfloat16), ihw_ref[...], (((0,), (0,)), ((), ())),
        preferred_element_type=jnp.float32)


@jax.jit
def kernel(x, exc, g_exc_x_w, g_exc_x_b, ln_e_x_g, ln_e_x_b,
           g_exc_e_w, g_exc_e_b, ln_e_e_g, ln_e_e_b,
           w_exc_x_w, w_exc_x_b, w_exc_ee_w, w_exc_ee_b,
           ln_out_e_g, ln_out_e_b):
    N, Cin, H, W = x.shape
    Ch = exc.shape[1]
    HW = H * W
    MT = 256
    K3 = Cin + Ch

    x_t = jnp.transpose(x.reshape(N, Cin, HW), (0, 2, 1))
    e_t = jnp.transpose(exc.reshape(N, Ch, HW), (0, 2, 1))
    xe = jnp.concatenate([x_t, e_t], axis=2).astype(jnp.bfloat16)
    ihw = jnp.eye(HW, dtype=jnp.bfloat16)

    wgx = g_exc_x_w.reshape(Ch, Cin).T
    wge = g_exc_e_w.reshape(Ch, Ch).T
    wwx = w_exc_x_w.reshape(Ch, Cin).T
    z_ec = jnp.zeros((Ch, Ch), jnp.float32)
    z_xc = jnp.zeros((Cin, Ch), jnp.float32)
    w3 = jnp.concatenate([
        jnp.concatenate([wgx, z_xc, wwx], axis=1),
        jnp.concatenate([z_ec, wge, z_ec], axis=1),
    ], axis=0).astype(jnp.bfloat16)
    bg = jnp.concatenate([g_exc_x_b, g_exc_e_b]).reshape(1, 2 * Ch)
    bwe = (w_exc_x_b + w_exc_ee_b).reshape(1, Ch)

    w7 = jnp.transpose(w_exc_ee_w, (2, 3, 1, 0)).reshape(
        _F, _F * Ch, Ch).astype(jnp.bfloat16)

    row = lambda v: v.reshape(1, Ch)

    def fixed(shape):
        n = len(shape)
        return pl.BlockSpec(shape, lambda b, _n=n: (0,) * _n)

    fn = pl.pallas_call(
        functools.partial(_cell_kernel, H, W, Ch, MT),
        out_shape=jax.ShapeDtypeStruct((N, Ch, HW), jnp.float32),
        grid=(N,),
        in_specs=[
            pl.BlockSpec((1, HW, K3), lambda b: (b, 0, 0)),
            fixed((HW, HW)),
            fixed((K3, 3 * Ch)),
            fixed((1, 2 * Ch)),
            fixed((_F, _F * Ch, Ch)),
            fixed((1, Ch)), fixed((1, Ch)),
            fixed((1, Ch)), fixed((1, Ch)),
            fixed((1, Ch)),
            fixed((1, Ch)), fixed((1, Ch)),
        ],
        out_specs=pl.BlockSpec((1, Ch, HW), lambda b: (b, 0, 0)),
        scratch_shapes=[
            pltpu.VMEM((HW + (_F - 1) * W, _F * Ch), jnp.bfloat16),
            pltpu.VMEM((HW, 3 * Ch), jnp.float32),
        ],
        compiler_params=pltpu.CompilerParams(
            dimension_semantics=("parallel",)),
    )
    out = fn(
        xe, ihw, w3, bg, w7,
        row(ln_e_x_g), row(ln_e_x_b),
        row(ln_e_e_g), row(ln_e_e_b),
        bwe,
        row(ln_out_e_g), row(ln_out_e_b),
    )
    return out.reshape(N, Ch, H, W)


# speedup vs baseline: 1.1497x; 1.1497x over previous
import functools

import jax
import jax.numpy as jnp
from jax.experimental import pallas as pl
from jax.experimental.pallas import tpu as pltpu

_F = 7
_P = (_F - 1) // 2
_EPS = 1e-5

def _cell_kernel(
    H, W, Ch, MT,
    xe_ref,
    e_ref,
    w3_ref,
    bg_ref,
    w7_ref,
    g1g_ref, g1b_ref,
    g2g_ref, g2b_ref,
    bwe_ref,
    g3g_ref, g3b_ref,
    out_ref,
    ew_ref,
    big_ref,
):
    HW = H * W
    PW = _P * W
    inv_n = 1.0 / float(HW * Ch)
    xe = xe_ref[0]

    big_ref[...] = jnp.dot(xe, w3_ref[...],
                           preferred_element_type=jnp.float32)

    bg = bg_ref[...]
    v1 = big_ref[:, 0:Ch]
    v2 = big_ref[:, Ch:2 * Ch]
    s1c = jnp.sum(v1, axis=0, keepdims=True)
    q1c = jnp.sum(v1 * v1, axis=0, keepdims=True)
    s2c = jnp.sum(v2, axis=0, keepdims=True)
    q2c = jnp.sum(v2 * v2, axis=0, keepdims=True)
    b1 = bg[:, 0:Ch]
    b2 = bg[:, Ch:2 * Ch]
    s1 = jnp.sum(s1c + float(HW) * b1, axis=1, keepdims=True)
    s2 = jnp.sum(s2c + float(HW) * b2, axis=1, keepdims=True)
    q1 = jnp.sum(q1c + 2.0 * b1 * s1c + float(HW) * b1 * b1,
                 axis=1, keepdims=True)
    q2 = jnp.sum(q2c + 2.0 * b2 * s2c + float(HW) * b2 * b2,
                 axis=1, keepdims=True)
    mu1 = s1 * inv_n
    var1 = q1 * inv_n - mu1 * mu1
    r1 = jax.lax.rsqrt(var1 + _EPS)
    mu2 = s2 * inv_n
    var2 = q2 * inv_n - mu2 * mu2
    r2 = jax.lax.rsqrt(var2 + _EPS)
    c1 = r1 * g1g_ref[...]
    c2 = r2 * g2g_ref[...]
    o12 = (g1b_ref[...] + g2b_ref[...]
           + (b1 - mu1) * c1 + (b2 - mu2) * c2)

    ew_ref[0:PW, :] = jnp.zeros((PW, _F * Ch), jnp.bfloat16)
    ew_ref[PW + HW:, :] = jnp.zeros((PW, _F * Ch), jnp.bfloat16)

    e2 = e_ref[0]
    pcol = jax.lax.broadcasted_iota(jnp.int32, (HW, Ch), 0) & (W - 1)
    for kw in range(_F):
        d = kw - _P
        rolled = pltpu.roll(e2, (-d) % HW, axis=0) if d else e2
        if d > 0:
            blk = jnp.where(pcol <= (W - 1 - d), rolled, 0.0)
        elif d < 0:
            blk = jnp.where(pcol >= (-d), rolled, 0.0)
        else:
            blk = rolled
        ew_ref[PW:PW + HW, kw * Ch:(kw + 1) * Ch] = blk.astype(jnp.bfloat16)

    bwe = bwe_ref[...]
    s3 = jnp.zeros((1, Ch), jnp.float32)
    q3 = jnp.zeros((1, Ch), jnp.float32)
    for m0 in range(0, HW, MT):
        acc = jnp.dot(ew_ref[m0:m0 + MT, :], w7_ref[0],
                      preferred_element_type=jnp.float32)
        for kh in range(1, _F):
            acc = acc + jnp.dot(ew_ref[m0 + kh * W:m0 + kh * W + MT, :],
                                w7_ref[kh], preferred_element_type=jnp.float32)
        cand = jnp.maximum(acc + big_ref[m0:m0 + MT, 2 * Ch:3 * Ch] + bwe, 0.0)
        g = jax.nn.sigmoid(big_ref[m0:m0 + MT, 0:Ch] * c1
                           + big_ref[m0:m0 + MT, Ch:2 * Ch] * c2 + o12)
        et = e2[m0:m0 + MT, :]
        pre = g * cand + et - g * et
        big_ref[m0:m0 + MT, 0:Ch] = pre
        s3 = s3 + jnp.sum(pre, axis=0, keepdims=True)
        q3 = q3 + jnp.sum(pre * pre, axis=0, keepdims=True)

    s3t = jnp.sum(s3, axis=1, keepdims=True)
    q3t = jnp.sum(q3, axis=1, keepdims=True)
    mu3 = s3t * inv_n
    var3 = q3t * inv_n - mu3 * mu3
    r3 = jax.lax.rsqrt(var3 + _EPS)
    c3 = r3 * g3g_ref[...]
    o3 = g3b_ref[...] - mu3 * c3
    out_ref[0] = jnp.maximum(big_ref[:, 0:Ch] * c3 + o3, 0.0)


@jax.jit
def kernel(x, exc, g_exc_x_w, g_exc_x_b, ln_e_x_g, ln_e_x_b,
           g_exc_e_w, g_exc_e_b, ln_e_e_g, ln_e_e_b,
           w_exc_x_w, w_exc_x_b, w_exc_ee_w, w_exc_ee_b,
           ln_out_e_g, ln_out_e_b):
    N, Cin, H, W = x.shape
    Ch = exc.shape[1]
    HW = H * W
    MT = 256
    K3 = Cin + Ch

    x_t = jnp.transpose(x.reshape(N, Cin, HW), (0, 2, 1))
    e_t = jnp.transpose(exc.reshape(N, Ch, HW), (0, 2, 1))
    xe = jnp.concatenate([x_t, e_t], axis=2).astype(jnp.bfloat16)

    wgx = g_exc_x_w.reshape(Ch, Cin).T
    wge = g_exc_e_w.reshape(Ch, Ch).T
    wwx = w_exc_x_w.reshape(Ch, Cin).T
    z_ec = jnp.zeros((Ch, Ch), jnp.float32)
    z_xc = jnp.zeros((Cin, Ch), jnp.float32)
    w3 = jnp.concatenate([
        jnp.concatenate([wgx, z_xc, wwx], axis=1),
        jnp.concatenate([z_ec, wge, z_ec], axis=1),
    ], axis=0).astype(jnp.bfloat16)
    bg = jnp.concatenate([g_exc_x_b, g_exc_e_b]).reshape(1, 2 * Ch)
    bwe = (w_exc_x_b + w_exc_ee_b).reshape(1, Ch)

    w7 = jnp.transpose(w_exc_ee_w, (2, 3, 1, 0)).reshape(
        _F, _F * Ch, Ch).astype(jnp.bfloat16)

    row = lambda v: v.reshape(1, Ch)

    def fixed(shape):
        n = len(shape)
        return pl.BlockSpec(shape, lambda b, _n=n: (0,) * _n)

    fn = pl.pallas_call(
        functools.partial(_cell_kernel, H, W, Ch, MT),
        out_shape=jax.ShapeDtypeStruct((N, HW, Ch), jnp.float32),
        grid=(N,),
        in_specs=[
            pl.BlockSpec((1, HW, K3), lambda b: (b, 0, 0)),
            pl.BlockSpec((1, HW, Ch), lambda b: (b, 0, 0)),
            fixed((K3, 3 * Ch)),
            fixed((1, 2 * Ch)),
            fixed((_F, _F * Ch, Ch)),
            fixed((1, Ch)), fixed((1, Ch)),
            fixed((1, Ch)), fixed((1, Ch)),
            fixed((1, Ch)),
            fixed((1, Ch)), fixed((1, Ch)),
        ],
        out_specs=pl.BlockSpec((1, HW, Ch), lambda b: (b, 0, 0)),
        scratch_shapes=[
            pltpu.VMEM((HW + (_F - 1) * W, _F * Ch), jnp.bfloat16),
            pltpu.VMEM((HW, 3 * Ch), jnp.float32),
        ],
        compiler_params=pltpu.CompilerParams(
            dimension_semantics=("parallel",)),
    )
    out = fn(
        xe, e_t, w3, bg, w7,
        row(ln_e_x_g), row(ln_e_x_b),
        row(ln_e_e_g), row(ln_e_e_b),
        bwe,
        row(ln_out_e_g), row(ln_out_e_b),
    )
    return jnp.transpose(out, (0, 2, 1)).reshape(N, Ch, H, W)


# speedup vs baseline: 1.1640x; 1.0124x over previous
import functools

import jax
import jax.numpy as jnp
from jax.experimental import pallas as pl
from jax.experimental.pallas import tpu as pltpu

_F = 7
_P = (_F - 1) // 2
_EPS = 1e-5

def _cell_kernel(
    H, W, Ch, MT, SB,
    xe_ref,
    e_ref,
    w3_ref,
    bg_ref,
    w7_ref,
    g1g_ref, g1b_ref,
    g2g_ref, g2b_ref,
    bwe_ref,
    g3g_ref, g3b_ref,
    out_ref,
    ew_ref,
    big_ref,
):
    HW = H * W
    PW = _P * W
    inv_n = 1.0 / float(HW * Ch)
    for s in range(SB):
        _one_sample(H, W, Ch, MT, xe_ref.at[s], e_ref.at[s], w3_ref, bg_ref,
                    w7_ref, g1g_ref, g1b_ref, g2g_ref, g2b_ref, bwe_ref,
                    g3g_ref, g3b_ref, out_ref.at[s], ew_ref.at[s],
                    big_ref.at[s])


def _one_sample(H, W, Ch, MT, xe_ref, e_ref, w3_ref, bg_ref, w7_ref,
                g1g_ref, g1b_ref, g2g_ref, g2b_ref, bwe_ref,
                g3g_ref, g3b_ref, out_ref, ew_ref, big_ref):
    HW = H * W
    PW = _P * W
    inv_n = 1.0 / float(HW * Ch)
    xe = xe_ref[...]

    big_ref[...] = jnp.dot(xe, w3_ref[...],
                           preferred_element_type=jnp.float32)

    bg = bg_ref[...]
    v1 = big_ref[:, 0:Ch]
    v2 = big_ref[:, Ch:2 * Ch]
    s1c = jnp.sum(v1, axis=0, keepdims=True)
    q1c = jnp.sum(v1 * v1, axis=0, keepdims=True)
    s2c = jnp.sum(v2, axis=0, keepdims=True)
    q2c = jnp.sum(v2 * v2, axis=0, keepdims=True)
    b1 = bg[:, 0:Ch]
    b2 = bg[:, Ch:2 * Ch]
    s1 = jnp.sum(s1c + float(HW) * b1, axis=1, keepdims=True)
    s2 = jnp.sum(s2c + float(HW) * b2, axis=1, keepdims=True)
    q1 = jnp.sum(q1c + 2.0 * b1 * s1c + float(HW) * b1 * b1,
                 axis=1, keepdims=True)
    q2 = jnp.sum(q2c + 2.0 * b2 * s2c + float(HW) * b2 * b2,
                 axis=1, keepdims=True)
    mu1 = s1 * inv_n
    var1 = q1 * inv_n - mu1 * mu1
    r1 = jax.lax.rsqrt(var1 + _EPS)
    mu2 = s2 * inv_n
    var2 = q2 * inv_n - mu2 * mu2
    r2 = jax.lax.rsqrt(var2 + _EPS)
    c1 = r1 * g1g_ref[...]
    c2 = r2 * g2g_ref[...]
    o12 = (g1b_ref[...] + g2b_ref[...]
           + (b1 - mu1) * c1 + (b2 - mu2) * c2)

    ew_ref[0:PW, :] = jnp.zeros((PW, _F * Ch), jnp.bfloat16)
    ew_ref[PW + HW:, :] = jnp.zeros((PW, _F * Ch), jnp.bfloat16)

    e2 = e_ref[...]
    pcol = jax.lax.broadcasted_iota(jnp.int32, (HW, Ch), 0) & (W - 1)
    for kw in range(_F):
        d = kw - _P
        rolled = pltpu.roll(e2, (-d) % HW, axis=0) if d else e2
        if d > 0:
            blk = jnp.where(pcol <= (W - 1 - d), rolled, 0.0)
        elif d < 0:
            blk = jnp.where(pcol >= (-d), rolled, 0.0)
        else:
            blk = rolled
        ew_ref[PW:PW + HW, kw * Ch:(kw + 1) * Ch] = blk.astype(jnp.bfloat16)

    bwe = bwe_ref[...]
    s3 = jnp.zeros((1, Ch), jnp.float32)
    q3 = jnp.zeros((1, Ch), jnp.float32)
    for m0 in range(0, HW, MT):
        acc = jnp.dot(ew_ref[m0:m0 + MT, :], w7_ref[0],
                      preferred_element_type=jnp.float32)
        for kh in range(1, _F):
            acc = acc + jnp.dot(ew_ref[m0 + kh * W:m0 + kh * W + MT, :],
                                w7_ref[kh], preferred_element_type=jnp.float32)
        cand = jnp.maximum(acc + big_ref[m0:m0 + MT, 2 * Ch:3 * Ch] + bwe, 0.0)
        g = jax.nn.sigmoid(big_ref[m0:m0 + MT, 0:Ch] * c1
                           + big_ref[m0:m0 + MT, Ch:2 * Ch] * c2 + o12)
        et = e2[m0:m0 + MT, :]
        pre = g * cand + et - g * et
        big_ref[m0:m0 + MT, 0:Ch] = pre
        s3 = s3 + jnp.sum(pre, axis=0, keepdims=True)
        q3 = q3 + jnp.sum(pre * pre, axis=0, keepdims=True)

    s3t = jnp.sum(s3, axis=1, keepdims=True)
    q3t = jnp.sum(q3, axis=1, keepdims=True)
    mu3 = s3t * inv_n
    var3 = q3t * inv_n - mu3 * mu3
    r3 = jax.lax.rsqrt(var3 + _EPS)
    c3 = r3 * g3g_ref[...]
    o3 = g3b_ref[...] - mu3 * c3
    out_ref[...] = jnp.maximum(big_ref[:, 0:Ch] * c3 + o3, 0.0)


@jax.jit
def kernel(x, exc, g_exc_x_w, g_exc_x_b, ln_e_x_g, ln_e_x_b,
           g_exc_e_w, g_exc_e_b, ln_e_e_g, ln_e_e_b,
           w_exc_x_w, w_exc_x_b, w_exc_ee_w, w_exc_ee_b,
           ln_out_e_g, ln_out_e_b):
    N, Cin, H, W = x.shape
    Ch = exc.shape[1]
    HW = H * W
    MT = 256
    SB = 2
    K3 = Cin + Ch

    x_t = jnp.transpose(x.reshape(N, Cin, HW), (0, 2, 1))
    e_t = jnp.transpose(exc.reshape(N, Ch, HW), (0, 2, 1))
    xe = jnp.concatenate([x_t, e_t], axis=2).astype(jnp.bfloat16)

    wgx = g_exc_x_w.reshape(Ch, Cin).T
    wge = g_exc_e_w.reshape(Ch, Ch).T
    wwx = w_exc_x_w.reshape(Ch, Cin).T
    z_ec = jnp.zeros((Ch, Ch), jnp.float32)
    z_xc = jnp.zeros((Cin, Ch), jnp.float32)
    w3 = jnp.concatenate([
        jnp.concatenate([wgx, z_xc, wwx], axis=1),
        jnp.concatenate([z_ec, wge, z_ec], axis=1),
    ], axis=0).astype(jnp.bfloat16)
    bg = jnp.concatenate([g_exc_x_b, g_exc_e_b]).reshape(1, 2 * Ch)
    bwe = (w_exc_x_b + w_exc_ee_b).reshape(1, Ch)

    w7 = jnp.transpose(w_exc_ee_w, (2, 3, 1, 0)).reshape(
        _F, _F * Ch, Ch).astype(jnp.bfloat16)

    row = lambda v: v.reshape(1, Ch)

    def fixed(shape):
        n = len(shape)
        return pl.BlockSpec(shape, lambda b, _n=n: (0,) * _n)

    fn = pl.pallas_call(
        functools.partial(_cell_kernel, H, W, Ch, MT, SB),
        out_shape=jax.ShapeDtypeStruct((N, HW, Ch), jnp.float32),
        grid=(N // SB,),
        in_specs=[
            pl.BlockSpec((SB, HW, K3), lambda b: (b, 0, 0)),
            pl.BlockSpec((SB, HW, Ch), lambda b: (b, 0, 0)),
            fixed((K3, 3 * Ch)),
            fixed((1, 2 * Ch)),
            fixed((_F, _F * Ch, Ch)),
            fixed((1, Ch)), fixed((1, Ch)),
            fixed((1, Ch)), fixed((1, Ch)),
            fixed((1, Ch)),
            fixed((1, Ch)), fixed((1, Ch)),
        ],
        out_specs=pl.BlockSpec((SB, HW, Ch), lambda b: (b, 0, 0)),
        scratch_shapes=[
            pltpu.VMEM((SB, HW + (_F - 1) * W, _F * Ch), jnp.bfloat16),
            pltpu.VMEM((SB, HW, 3 * Ch), jnp.float32),
        ],
        compiler_params=pltpu.CompilerParams(
            dimension_semantics=("parallel",)),
    )
    out = fn(
        xe, e_t, w3, bg, w7,
        row(ln_e_x_g), row(ln_e_x_b),
        row(ln_e_e_g), row(ln_e_e_b),
        bwe,
        row(ln_out_e_g), row(ln_out_e_b),
    )
    return jnp.transpose(out, (0, 2, 1)).reshape(N, Ch, H, W)


# speedup vs baseline: 1.1691x; 1.0044x over previous
import functools

import jax
import jax.numpy as jnp
from jax.experimental import pallas as pl
from jax.experimental.pallas import tpu as pltpu

_F = 7
_P = (_F - 1) // 2
_EPS = 1e-5

def _cell_kernel(
    H, W, Ch, MT, SB,
    xe_ref,
    e_ref,
    w3_ref,
    bg_ref,
    w7_ref,
    g1g_ref, g1b_ref,
    g2g_ref, g2b_ref,
    bwe_ref,
    g3g_ref, g3b_ref,
    out_ref,
    ew_ref,
    big_ref,
):
    HW = H * W
    PW = _P * W
    inv_n = 1.0 / float(HW * Ch)
    for s in range(SB):
        _one_sample(H, W, Ch, MT, xe_ref.at[s], e_ref.at[s], w3_ref, bg_ref,
                    w7_ref, g1g_ref, g1b_ref, g2g_ref, g2b_ref, bwe_ref,
                    g3g_ref, g3b_ref, out_ref.at[s], ew_ref.at[s],
                    big_ref.at[s])


def _one_sample(H, W, Ch, MT, xe_ref, e_ref, w3_ref, bg_ref, w7_ref,
                g1g_ref, g1b_ref, g2g_ref, g2b_ref, bwe_ref,
                g3g_ref, g3b_ref, out_ref, ew_ref, big_ref):
    HW = H * W
    PW = _P * W
    inv_n = 1.0 / float(HW * Ch)
    xe = xe_ref[...]

    big_ref[...] = jnp.dot(xe, w3_ref[...],
                           preferred_element_type=jnp.float32)

    bg = bg_ref[...]
    v1 = big_ref[:, 0:Ch]
    v2 = big_ref[:, Ch:2 * Ch]
    s1c = jnp.sum(v1, axis=0, keepdims=True)
    q1c = jnp.sum(v1 * v1, axis=0, keepdims=True)
    s2c = jnp.sum(v2, axis=0, keepdims=True)
    q2c = jnp.sum(v2 * v2, axis=0, keepdims=True)
    b1 = bg[:, 0:Ch]
    b2 = bg[:, Ch:2 * Ch]
    s1 = jnp.sum(s1c + float(HW) * b1, axis=1, keepdims=True)
    s2 = jnp.sum(s2c + float(HW) * b2, axis=1, keepdims=True)
    q1 = jnp.sum(q1c + 2.0 * b1 * s1c + float(HW) * b1 * b1,
                 axis=1, keepdims=True)
    q2 = jnp.sum(q2c + 2.0 * b2 * s2c + float(HW) * b2 * b2,
                 axis=1, keepdims=True)
    mu1 = s1 * inv_n
    var1 = q1 * inv_n - mu1 * mu1
    r1 = jax.lax.rsqrt(var1 + _EPS)
    mu2 = s2 * inv_n
    var2 = q2 * inv_n - mu2 * mu2
    r2 = jax.lax.rsqrt(var2 + _EPS)
    c1 = r1 * g1g_ref[...]
    c2 = r2 * g2g_ref[...]
    o12 = (g1b_ref[...] + g2b_ref[...]
           + (b1 - mu1) * c1 + (b2 - mu2) * c2)

    ew_ref[0:PW, :] = jnp.zeros((PW, _F * Ch), jnp.bfloat16)
    ew_ref[PW + HW:, :] = jnp.zeros((PW, _F * Ch), jnp.bfloat16)

    e2 = e_ref[...]
    pcol = jax.lax.broadcasted_iota(jnp.int32, (HW, Ch), 0) & (W - 1)
    for kw in range(_F):
        d = kw - _P
        rolled = pltpu.roll(e2, (-d) % HW, axis=0) if d else e2
        if d > 0:
            blk = jnp.where(pcol <= (W - 1 - d), rolled, 0.0)
        elif d < 0:
            blk = jnp.where(pcol >= (-d), rolled, 0.0)
        else:
            blk = rolled
        ew_ref[PW:PW + HW, kw * Ch:(kw + 1) * Ch] = blk.astype(jnp.bfloat16)

    bwe = bwe_ref[...]
    s3 = jnp.zeros((1, Ch), jnp.float32)
    q3 = jnp.zeros((1, Ch), jnp.float32)
    for m0 in range(0, HW, MT):
        acc = jnp.dot(ew_ref[m0:m0 + MT, :], w7_ref[0],
                      preferred_element_type=jnp.float32)
        for kh in range(1, _F):
            acc = acc + jnp.dot(ew_ref[m0 + kh * W:m0 + kh * W + MT, :],
                                w7_ref[kh], preferred_element_type=jnp.float32)
        cand = jnp.maximum(acc + big_ref[m0:m0 + MT, 2 * Ch:3 * Ch] + bwe, 0.0)
        g = jax.nn.sigmoid(big_ref[m0:m0 + MT, 0:Ch] * c1
                           + big_ref[m0:m0 + MT, Ch:2 * Ch] * c2 + o12)
        et = e2[m0:m0 + MT, :]
        pre = g * cand + et - g * et
        big_ref[m0:m0 + MT, 0:Ch] = pre
        s3 = s3 + jnp.sum(pre, axis=0, keepdims=True)
        q3 = q3 + jnp.sum(pre * pre, axis=0, keepdims=True)

    s3t = jnp.sum(s3, axis=1, keepdims=True)
    q3t = jnp.sum(q3, axis=1, keepdims=True)
    mu3 = s3t * inv_n
    var3 = q3t * inv_n - mu3 * mu3
    r3 = jax.lax.rsqrt(var3 + _EPS)
    c3 = r3 * g3g_ref[...]
    o3 = g3b_ref[...] - mu3 * c3
    out_ref[...] = jnp.maximum(big_ref[:, 0:Ch] * c3 + o3, 0.0)


@jax.jit
def kernel(x, exc, g_exc_x_w, g_exc_x_b, ln_e_x_g, ln_e_x_b,
           g_exc_e_w, g_exc_e_b, ln_e_e_g, ln_e_e_b,
           w_exc_x_w, w_exc_x_b, w_exc_ee_w, w_exc_ee_b,
           ln_out_e_g, ln_out_e_b):
    N, Cin, H, W = x.shape
    Ch = exc.shape[1]
    HW = H * W
    MT = 256
    SB = 4
    K3 = Cin + Ch

    x_t = jnp.transpose(x.reshape(N, Cin, HW), (0, 2, 1))
    e_t = jnp.transpose(exc.reshape(N, Ch, HW), (0, 2, 1))
    xe = jnp.concatenate([x_t, e_t], axis=2).astype(jnp.bfloat16)

    wgx = g_exc_x_w.reshape(Ch, Cin).T
    wge = g_exc_e_w.reshape(Ch, Ch).T
    wwx = w_exc_x_w.reshape(Ch, Cin).T
    z_ec = jnp.zeros((Ch, Ch), jnp.float32)
    z_xc = jnp.zeros((Cin, Ch), jnp.float32)
    w3 = jnp.concatenate([
        jnp.concatenate([wgx, z_xc, wwx], axis=1),
        jnp.concatenate([z_ec, wge, z_ec], axis=1),
    ], axis=0).astype(jnp.bfloat16)
    bg = jnp.concatenate([g_exc_x_b, g_exc_e_b]).reshape(1, 2 * Ch)
    bwe = (w_exc_x_b + w_exc_ee_b).reshape(1, Ch)

    w7 = jnp.transpose(w_exc_ee_w, (2, 3, 1, 0)).reshape(
        _F, _F * Ch, Ch).astype(jnp.bfloat16)

    row = lambda v: v.reshape(1, Ch)

    def fixed(shape):
        n = len(shape)
        return pl.BlockSpec(shape, lambda b, _n=n: (0,) * _n)

    fn = pl.pallas_call(
        functools.partial(_cell_kernel, H, W, Ch, MT, SB),
        out_shape=jax.ShapeDtypeStruct((N, HW, Ch), jnp.float32),
        grid=(N // SB,),
        in_specs=[
            pl.BlockSpec((SB, HW, K3), lambda b: (b, 0, 0)),
            pl.BlockSpec((SB, HW, Ch), lambda b: (b, 0, 0)),
            fixed((K3, 3 * Ch)),
            fixed((1, 2 * Ch)),
            fixed((_F, _F * Ch, Ch)),
            fixed((1, Ch)), fixed((1, Ch)),
            fixed((1, Ch)), fixed((1, Ch)),
            fixed((1, Ch)),
            fixed((1, Ch)), fixed((1, Ch)),
        ],
        out_specs=pl.BlockSpec((SB, HW, Ch), lambda b: (b, 0, 0)),
        scratch_shapes=[
            pltpu.VMEM((SB, HW + (_F - 1) * W, _F * Ch), jnp.bfloat16),
            pltpu.VMEM((SB, HW, 3 * Ch), jnp.float32),
        ],
        compiler_params=pltpu.CompilerParams(
            dimension_semantics=("parallel",)),
    )
    out = fn(
        xe, e_t, w3, bg, w7,
        row(ln_e_x_g), row(ln_e_x_b),
        row(ln_e_e_g), row(ln_e_e_b),
        bwe,
        row(ln_out_e_g), row(ln_out_e_b),
    )
    return jnp.transpose(out, (0, 2, 1)).reshape(N, Ch, H, W)


# speedup vs baseline: 1.2115x; 1.0363x over previous
import functools

import jax
import jax.numpy as jnp
from jax.experimental import pallas as pl
from jax.experimental.pallas import tpu as pltpu

_F = 7
_P = (_F - 1) // 2
_EPS = 1e-5

def _cell_kernel(
    H, W, Ch, MT, SB,
    xe_ref,
    w3_ref,
    bg_ref,
    w7_ref,
    g1g_ref, g1b_ref,
    g2g_ref, g2b_ref,
    bwe_ref,
    g3g_ref, g3b_ref,
    out_ref,
    ew_ref,
    big_ref,
):
    HW = H * W
    PW = _P * W
    inv_n = 1.0 / float(HW * Ch)
    Cin = xe_ref.shape[2] - Ch
    for s in range(SB):
        _one_sample(H, W, Ch, Cin, MT, xe_ref.at[s], w3_ref, bg_ref,
                    w7_ref, g1g_ref, g1b_ref, g2g_ref, g2b_ref, bwe_ref,
                    g3g_ref, g3b_ref, out_ref.at[s], ew_ref.at[s],
                    big_ref.at[s])


def _one_sample(H, W, Ch, Cin, MT, xe_ref, w3_ref, bg_ref, w7_ref,
                g1g_ref, g1b_ref, g2g_ref, g2b_ref, bwe_ref,
                g3g_ref, g3b_ref, out_ref, ew_ref, big_ref):
    HW = H * W
    PW = _P * W
    inv_n = 1.0 / float(HW * Ch)
    xe = xe_ref[...]

    big_ref[...] = jnp.dot(xe, w3_ref[...],
                           preferred_element_type=jnp.float32)

    bg = bg_ref[...]
    v1 = big_ref[:, 0:Ch]
    v2 = big_ref[:, Ch:2 * Ch]
    s1c = jnp.sum(v1, axis=0, keepdims=True)
    q1c = jnp.sum(v1 * v1, axis=0, keepdims=True)
    s2c = jnp.sum(v2, axis=0, keepdims=True)
    q2c = jnp.sum(v2 * v2, axis=0, keepdims=True)
    b1 = bg[:, 0:Ch]
    b2 = bg[:, Ch:2 * Ch]
    s1 = jnp.sum(s1c + float(HW) * b1, axis=1, keepdims=True)
    s2 = jnp.sum(s2c + float(HW) * b2, axis=1, keepdims=True)
    q1 = jnp.sum(q1c + 2.0 * b1 * s1c + float(HW) * b1 * b1,
                 axis=1, keepdims=True)
    q2 = jnp.sum(q2c + 2.0 * b2 * s2c + float(HW) * b2 * b2,
                 axis=1, keepdims=True)
    mu1 = s1 * inv_n
    var1 = q1 * inv_n - mu1 * mu1
    r1 = jax.lax.rsqrt(var1 + _EPS)
    mu2 = s2 * inv_n
    var2 = q2 * inv_n - mu2 * mu2
    r2 = jax.lax.rsqrt(var2 + _EPS)
    c1 = r1 * g1g_ref[...]
    c2 = r2 * g2g_ref[...]
    o12 = (g1b_ref[...] + g2b_ref[...]
           + (b1 - mu1) * c1 + (b2 - mu2) * c2)

    ew_ref[0:PW, :] = jnp.zeros((PW, _F * Ch), jnp.bfloat16)
    ew_ref[PW + HW:, :] = jnp.zeros((PW, _F * Ch), jnp.bfloat16)

    e2 = xe[:, Cin:Cin + Ch].astype(jnp.float32)
    pcol = jax.lax.broadcasted_iota(jnp.int32, (HW, Ch), 0) & (W - 1)
    for kw in range(_F):
        d = kw - _P
        rolled = pltpu.roll(e2, (-d) % HW, axis=0) if d else e2
        if d > 0:
            blk = jnp.where(pcol <= (W - 1 - d), rolled, 0.0)
        elif d < 0:
            blk = jnp.where(pcol >= (-d), rolled, 0.0)
        else:
            blk = rolled
        ew_ref[PW:PW + HW, kw * Ch:(kw + 1) * Ch] = blk.astype(jnp.bfloat16)

    bwe = bwe_ref[...]
    s3 = jnp.zeros((1, Ch), jnp.float32)
    q3 = jnp.zeros((1, Ch), jnp.float32)
    for m0 in range(0, HW, MT):
        acc = jnp.dot(ew_ref[m0:m0 + MT, :], w7_ref[0],
                      preferred_element_type=jnp.float32)
        for kh in range(1, _F):
            acc = acc + jnp.dot(ew_ref[m0 + kh * W:m0 + kh * W + MT, :],
                                w7_ref[kh], preferred_element_type=jnp.float32)
        cand = jnp.maximum(acc + big_ref[m0:m0 + MT, 2 * Ch:3 * Ch] + bwe, 0.0)
        g = jax.nn.sigmoid(big_ref[m0:m0 + MT, 0:Ch] * c1
                           + big_ref[m0:m0 + MT, Ch:2 * Ch] * c2 + o12)
        et = e2[m0:m0 + MT, :]
        pre = g * cand + et - g * et
        big_ref[m0:m0 + MT, 0:Ch] = pre
        s3 = s3 + jnp.sum(pre, axis=0, keepdims=True)
        q3 = q3 + jnp.sum(pre * pre, axis=0, keepdims=True)

    s3t = jnp.sum(s3, axis=1, keepdims=True)
    q3t = jnp.sum(q3, axis=1, keepdims=True)
    mu3 = s3t * inv_n
    var3 = q3t * inv_n - mu3 * mu3
    r3 = jax.lax.rsqrt(var3 + _EPS)
    c3 = r3 * g3g_ref[...]
    o3 = g3b_ref[...] - mu3 * c3
    out_ref[...] = jnp.maximum(big_ref[:, 0:Ch] * c3 + o3, 0.0)


@jax.jit
def kernel(x, exc, g_exc_x_w, g_exc_x_b, ln_e_x_g, ln_e_x_b,
           g_exc_e_w, g_exc_e_b, ln_e_e_g, ln_e_e_b,
           w_exc_x_w, w_exc_x_b, w_exc_ee_w, w_exc_ee_b,
           ln_out_e_g, ln_out_e_b):
    N, Cin, H, W = x.shape
    Ch = exc.shape[1]
    HW = H * W
    MT = 256
    SB = max(v for v in (4, 2, 1) if N % v == 0)
    K3 = Cin + Ch

    x_t = jnp.transpose(x.reshape(N, Cin, HW), (0, 2, 1))
    e_t = jnp.transpose(exc.reshape(N, Ch, HW), (0, 2, 1))
    xe = jnp.concatenate([x_t, e_t], axis=2).astype(jnp.bfloat16)

    wgx = g_exc_x_w.reshape(Ch, Cin).T
    wge = g_exc_e_w.reshape(Ch, Ch).T
    wwx = w_exc_x_w.reshape(Ch, Cin).T
    z_ec = jnp.zeros((Ch, Ch), jnp.float32)
    z_xc = jnp.zeros((Cin, Ch), jnp.float32)
    w3 = jnp.concatenate([
        jnp.concatenate([wgx, z_xc, wwx], axis=1),
        jnp.concatenate([z_ec, wge, z_ec], axis=1),
    ], axis=0).astype(jnp.bfloat16)
    bg = jnp.concatenate([g_exc_x_b, g_exc_e_b]).reshape(1, 2 * Ch)
    bwe = (w_exc_x_b + w_exc_ee_b).reshape(1, Ch)

    w7 = jnp.transpose(w_exc_ee_w, (2, 3, 1, 0)).reshape(
        _F, _F * Ch, Ch).astype(jnp.bfloat16)

    row = lambda v: v.reshape(1, Ch)

    def fixed(shape):
        n = len(shape)
        return pl.BlockSpec(shape, lambda b, _n=n: (0,) * _n)

    fn = pl.pallas_call(
        functools.partial(_cell_kernel, H, W, Ch, MT, SB),
        out_shape=jax.ShapeDtypeStruct((N, HW, Ch), jnp.float32),
        grid=(N // SB,),
        in_specs=[
            pl.BlockSpec((SB, HW, K3), lambda b: (b, 0, 0)),
            fixed((K3, 3 * Ch)),
            fixed((1, 2 * Ch)),
            fixed((_F, _F * Ch, Ch)),
            fixed((1, Ch)), fixed((1, Ch)),
            fixed((1, Ch)), fixed((1, Ch)),
            fixed((1, Ch)),
            fixed((1, Ch)), fixed((1, Ch)),
        ],
        out_specs=pl.BlockSpec((SB, HW, Ch), lambda b: (b, 0, 0)),
        scratch_shapes=[
            pltpu.VMEM((SB, HW + (_F - 1) * W, _F * Ch), jnp.bfloat16),
            pltpu.VMEM((SB, HW, 3 * Ch), jnp.float32),
        ],
        compiler_params=pltpu.CompilerParams(
            dimension_semantics=("parallel",)),
    )
    out = fn(
        xe, w3, bg, w7,
        row(ln_e_x_g), row(ln_e_x_b),
        row(ln_e_e_g), row(ln_e_e_b),
        bwe,
        row(ln_out_e_g), row(ln_out_e_b),
    )
    return jnp.transpose(out, (0, 2, 1)).reshape(N, Ch, H, W)
